# Initial kernel scaffold; baseline (speedup 1.0000x reference)
import jax
import jax.numpy as jnp
from jax.experimental import pallas as pl


def kernel(node_feature, edge_index, edge_type, batch, center_mask, reaction, W_flag, b_flag, W_self0, b_self0, W_rel0, W_self1, b_self1, W_rel1, W_self2, b_self2, W_rel2):
    raise NotImplementedError("write your pallas kernel here")


# speedup vs baseline: 4.5101x; 4.5101x over previous
import jax
import jax.numpy as jnp
from jax import lax
from jax.experimental import pallas as pl
from jax.experimental.pallas import tpu as pltpu
from jax.experimental.pallas import tpu_sc as plsc

N_ = 10000
E_ = 320000
D_ = 128
H_ = 64
G_ = 64
R_ = 4
NREACT_ = 10

NC_ = 2
NS_ = 16
NW_ = NC_ * NS_
L_ = 16

C_ = 128
NCHUNK_ = -(-E_ // (NW_ * C_))
E_PAD_ = NW_ * NCHUNK_ * C_
W_ = 128
NPAD_ = 10240
RPT_ = NPAD_ // NS_
ZB_ = 128

BN_ = 1000


def _gidx_body(s_ref, t_ref, o_ref):
    o_ref[...] = t_ref[...] * N_ + s_ref[...]


def _gidx(src2, typ2):
    return pl.pallas_call(
        _gidx_body,
        out_shape=jax.ShapeDtypeStruct((E_PAD_ // C_, C_), jnp.int32),
    )(src2, typ2)


def _prep_body(nf_ref, cm_ref, wf_ref, bf_ref, ni_ref):
    ni_ref[...] = nf_ref[...] + cm_ref[...] * wf_ref[1:2, :] + bf_ref[...]


def _prep(node_feature, cm, w_flag, b_flag2):
    return pl.pallas_call(
        _prep_body,
        grid=(N_ // BN_,),
        in_specs=[
            pl.BlockSpec((BN_, D_), lambda i: (i, 0)),
            pl.BlockSpec((BN_, 1), lambda i: (i, 0)),
            pl.BlockSpec((2, D_), lambda i: (0, 0)),
            pl.BlockSpec((1, D_), lambda i: (0, 0)),
        ],
        out_specs=pl.BlockSpec((BN_, D_), lambda i: (i, 0)),
        out_shape=jax.ShapeDtypeStruct((N_, D_), jnp.float32),
    )(node_feature, cm, w_flag, b_flag2)


def _relmm_body(x_ref, w_ref, o_ref):
    f32 = jnp.float32
    y = jnp.dot(x_ref[...], w_ref[0], preferred_element_type=f32)
    tail = jnp.concatenate(
        [jnp.ones((BN_, 1), f32), jnp.zeros((BN_, W_ - H_ - 1), f32)], axis=1)
    o_ref[...] = jnp.concatenate([y, tail], axis=1)


def _relmm(x, w_rel):
    din = x.shape[1]
    return pl.pallas_call(
        _relmm_body,
        grid=(R_, N_ // BN_),
        in_specs=[
            pl.BlockSpec((BN_, din), lambda r, i: (i, 0)),
            pl.BlockSpec((1, din, H_), lambda r, i: (r, 0, 0)),
        ],
        out_specs=pl.BlockSpec((BN_, W_), lambda r, i: (r * (N_ // BN_) + i, 0)),
        out_shape=jax.ShapeDtypeStruct((R_ * N_, W_), jnp.float32),
    )(x, w_rel)


def _sc_edge_body(y_hbm, gi_hbm, di_hbm, part_hbm,
                  gi_v, di_v, rows_v, acc_sh, sem):
    cid = lax.axis_index("c")
    sid = lax.axis_index("s")
    wid = sid * NC_ + cid

    def zfill(k, c):
        rows_v[k >> 3, pl.ds((k & 7) * L_, L_)] = jnp.zeros((L_,), jnp.float32)
        return c
    lax.fori_loop(0, ZB_ * (W_ // L_), zfill, 0)
    for b in range(RPT_ // ZB_):
        pltpu.sync_copy(rows_v, acc_sh.at[pl.ds(sid * RPT_ + b * ZB_, ZB_)])

    pltpu.sync_copy(gi_hbm.at[wid], gi_v)
    pltpu.sync_copy(di_hbm.at[wid], di_v)

    plsc.subcore_barrier()

    def chunk(j, c):
        pltpu.async_copy(y_hbm.at[gi_v.at[j]], rows_v, sem).wait()
        pltpu.sync_copy(rows_v, acc_sh.at[di_v.at[j]], add=True)
        return c
    lax.fori_loop(0, NCHUNK_, chunk, 0)

    plsc.subcore_barrier()

    pltpu.sync_copy(acc_sh.at[pl.ds(sid * RPT_, RPT_)],
                    part_hbm.at[cid, pl.ds(sid * RPT_, RPT_)])


_sc_pass = pl.kernel(
    _sc_edge_body,
    out_type=jax.ShapeDtypeStruct((NC_, NPAD_, W_), jnp.float32),
    mesh=plsc.VectorSubcoreMesh(core_axis_name="c", subcore_axis_name="s"),
    scratch_types=[
        pltpu.VMEM((NCHUNK_, C_), jnp.int32),
        pltpu.VMEM((NCHUNK_, C_), jnp.int32),
        pltpu.VMEM((C_, W_), jnp.float32),
        pltpu.VMEM_SHARED((NPAD_, W_), jnp.float32),
        pltpu.SemaphoreType.DMA,
    ],
)


def _layer_body(x_ref, p_ref, ws_ref, bs_ref, h_ref):
    p = p_ref[0, :, 0:H_] + p_ref[1, :, 0:H_]
    dg = p_ref[0, :, H_:H_ + 1] + p_ref[1, :, H_:H_ + 1]
    agg = p / jnp.maximum(dg, 1.0)
    h_ref[...] = jnp.maximum(
        jnp.dot(x_ref[...], ws_ref[...], preferred_element_type=jnp.float32)
        + bs_ref[...] + agg, 0.0)


def _layer(x, part, w_self, b_self2):
    din = x.shape[1]
    return pl.pallas_call(
        _layer_body,
        grid=(N_ // BN_,),
        in_specs=[
            pl.BlockSpec((BN_, din), lambda i: (i, 0)),
            pl.BlockSpec((NC_, BN_, W_), lambda i: (0, i, 0)),
            pl.BlockSpec((din, H_), lambda i: (0, 0)),
            pl.BlockSpec((1, H_), lambda i: (0, 0)),
        ],
        out_specs=pl.BlockSpec((BN_, H_), lambda i: (i, 0)),
        out_shape=jax.ShapeDtypeStruct((N_, H_), jnp.float32),
    )(x, part, w_self, b_self2)


def _pool_body(h1_ref, h2_ref, h3_ref, b_ref, r_ref, gc_ref):
    f32 = jnp.float32
    nf = jnp.concatenate([h1_ref[...], h2_ref[...], h3_ref[...]], axis=1)
    bot = (b_ref[...] == lax.broadcasted_iota(jnp.int32, (G_, N_), 0)).astype(f32)
    seg = jnp.dot(bot, nf, preferred_element_type=f32)
    cnt = jnp.sum(bot, axis=1).reshape(G_, 1)
    gf = seg / jnp.maximum(cnt, 1.0)
    oh = (r_ref[...] == lax.broadcasted_iota(jnp.int32, (G_, NREACT_), 1)).astype(f32)
    gc_ref[...] = jnp.concatenate([gf, oh], axis=1)


def _pool(h1, h2, h3, batch_row, reaction_col):
    return pl.pallas_call(
        _pool_body,
        out_shape=jax.ShapeDtypeStruct((G_, 3 * H_ + NREACT_), jnp.float32),
    )(h1, h2, h3, batch_row, reaction_col)


def _ctx_body(h1_ref, h2_ref, h3_ref, nf_ref, b_ref, gc_ref, out_ref):
    f32 = jnp.float32
    bo = (b_ref[...] == lax.broadcasted_iota(jnp.int32, (BN_, G_), 1)).astype(f32)
    gcb = jnp.dot(bo, gc_ref[...], preferred_element_type=f32)
    out_ref[...] = jnp.concatenate(
        [h1_ref[...], h2_ref[...], h3_ref[...], nf_ref[...], gcb], axis=1)


def _ctx(h1, h2, h3, node_feature, batch_col, gc):
    dctx = 3 * H_ + D_ + (3 * H_ + NREACT_)
    return pl.pallas_call(
        _ctx_body,
        grid=(N_ // BN_,),
        in_specs=[
            pl.BlockSpec((BN_, H_), lambda i: (i, 0)),
            pl.BlockSpec((BN_, H_), lambda i: (i, 0)),
            pl.BlockSpec((BN_, H_), lambda i: (i, 0)),
            pl.BlockSpec((BN_, D_), lambda i: (i, 0)),
            pl.BlockSpec((BN_, 1), lambda i: (i, 0)),
            pl.BlockSpec((G_, 3 * H_ + NREACT_), lambda i: (0, 0)),
        ],
        out_specs=pl.BlockSpec((BN_, dctx), lambda i: (i, 0)),
        out_shape=jax.ShapeDtypeStruct((N_, dctx), jnp.float32),
    )(h1, h2, h3, node_feature, batch_col, gc)


def kernel(node_feature, edge_index, edge_type, batch, center_mask, reaction,
           W_flag, b_flag, W_self0, b_self0, W_rel0, W_self1, b_self1, W_rel1,
           W_self2, b_self2, W_rel2):
    f32 = jnp.float32
    i32 = jnp.int32
    src = edge_index[0].astype(i32)
    dst = edge_index[1].astype(i32)
    typ = edge_type.astype(i32)
    pad = E_PAD_ - E_
    src_p = jnp.concatenate([src, jnp.zeros((pad,), i32)])
    typ_p = jnp.concatenate([typ, jnp.zeros((pad,), i32)])
    dst_p = jnp.concatenate([dst, jnp.full((pad,), N_, i32)])

    gidx3 = _gidx(src_p.reshape(E_PAD_ // C_, C_),
                  typ_p.reshape(E_PAD_ // C_, C_)).reshape(NW_, NCHUNK_, C_)
    dst3 = dst_p.reshape(NW_, NCHUNK_, C_)

    cm = center_mask.astype(f32).reshape(N_, 1)
    ni = _prep(node_feature, cm, W_flag, b_flag.reshape(1, D_))

    h = ni
    hs = []
    for w_rel, w_self, b_self in (
            (W_rel0, W_self0, b_self0),
            (W_rel1, W_self1, b_self1),
            (W_rel2, W_self2, b_self2)):
        y = _relmm(h, w_rel)
        part = _sc_pass(y, gidx3, dst3)
        h = _layer(h, part, w_self, b_self.reshape(1, H_))
        hs.append(h)
    h1, h2, h3 = hs

    gc = _pool(h1, h2, h3, batch.astype(i32).reshape(1, N_),
               reaction.astype(i32).reshape(G_, 1))
    nc = _ctx(h1, h2, h3, node_feature, batch.astype(i32).reshape(N_, 1), gc)
    return (nc, gc)
